# Initial kernel scaffold; baseline (speedup 1.0000x reference)
import jax
import jax.numpy as jnp
from jax.experimental import pallas as pl


def kernel(hidden_states, importance_scores, Wc, bc, Wi, bi, Wf, bf, Wdc, bdc, Wdi, bdi, Wdf, bdf, gamma, beta):
    raise NotImplementedError("write your pallas kernel here")


# speedup vs baseline: 1.3327x; 1.3327x over previous
import jax
import jax.numpy as jnp
from jax.experimental import pallas as pl

H = 2048
SEQ = 4096
BLK = 256


def _dense_body(s_ref, x_ref, wc_ref, bc_ref, wi_ref, bi_ref, wf_ref, bf_ref,
                wdc_ref, bdc_ref, wdi_ref, bdi_ref, wdf_ref, bdf_ref,
                g_ref, b_ref, o_ref):
    x = x_ref[...]
    f32 = jnp.float32

    def branch(w_ref, b1_ref, wd_ref, b2_ref):
        z = jnp.dot(x, w_ref[...], preferred_element_type=f32) + b1_ref[...]
        y = jnp.dot(z.astype(jnp.bfloat16), wd_ref[...],
                    preferred_element_type=f32) + b2_ref[...]
        return y

    yc = branch(wc_ref, bc_ref, wdc_ref, bdc_ref)
    yi = branch(wi_ref, bi_ref, wdi_ref, bdi_ref)
    yf = branch(wf_ref, bf_ref, wdf_ref, bdf_ref)
    s = s_ref[...]
    y = jnp.where(s > 0.8, yc, jnp.where(s > 0.4, yi, yf))
    mean = jnp.mean(y, axis=-1, keepdims=True)
    yc0 = y - mean
    var = jnp.mean(yc0 * yc0, axis=-1, keepdims=True)
    o_ref[...] = yc0 * jax.lax.rsqrt(var + 1e-5) * g_ref[...] + b_ref[...]


def kernel(hidden_states, importance_scores, Wc, bc, Wi, bi, Wf, bf,
           Wdc, bdc, Wdi, bdi, Wdf, bdf, gamma, beta):
    bf16 = jnp.bfloat16
    x = hidden_states.astype(bf16)
    s2 = importance_scores.reshape(SEQ, 1)
    full = lambda shape: pl.BlockSpec(shape, lambda i: (0, 0))
    row = lambda d: pl.BlockSpec((1, d), lambda i: (0, 0))

    out = pl.pallas_call(
        _dense_body,
        grid=(SEQ // BLK,),
        in_specs=[
            pl.BlockSpec((BLK, 1), lambda i: (i, 0)),
            pl.BlockSpec((BLK, H), lambda i: (i, 0)),
            full(Wc.shape), row(bc.shape[0]),
            full(Wi.shape), row(bi.shape[0]),
            full(Wf.shape), row(bf.shape[0]),
            full(Wdc.shape), row(H),
            full(Wdi.shape), row(H),
            full(Wdf.shape), row(H),
            row(H), row(H),
        ],
        out_specs=pl.BlockSpec((BLK, H), lambda i: (i, 0)),
        out_shape=jax.ShapeDtypeStruct((SEQ, H), jnp.float32),
    )(s2, x,
      Wc.astype(bf16), bc.reshape(1, -1),
      Wi.astype(bf16), bi.reshape(1, -1),
      Wf.astype(bf16), bf.reshape(1, -1),
      Wdc.astype(bf16), bdc.reshape(1, -1),
      Wdi.astype(bf16), bdi.reshape(1, -1),
      Wdf.astype(bf16), bdf.reshape(1, -1),
      gamma.reshape(1, -1), beta.reshape(1, -1))
    return out


# speedup vs baseline: 1.6038x; 1.2033x over previous
import functools

import jax
import jax.numpy as jnp
from jax import lax
from jax.experimental import pallas as pl
from jax.experimental.pallas import tpu as pltpu
from jax.experimental.pallas import tpu_sc as plsc

H = 2048
SEQ = 4096
BLK = 256
NBLK = SEQ // BLK
XS_ROWS = SEQ + 2 * BLK
ROWS_32 = SEQ // 32
CHUNK = 16

f32 = jnp.float32
bf16 = jnp.bfloat16
i32 = jnp.int32


def _routing_body(s_ref, pos_ref, meta_ref):
    s = s_ref[...]
    mc = (s > 0.8)
    mi = jnp.logical_and(s > 0.4, jnp.logical_not(mc))
    mf = jnp.logical_not(s > 0.4)

    lane = lax.broadcasted_iota(i32, (128, 128), 0)
    lane_t = lax.broadcasted_iota(i32, (128, 128), 1)
    triu_incl = (lane <= lane_t).astype(bf16)
    row = lax.broadcasted_iota(i32, (32, 32), 0)
    row_t = lax.broadcasted_iota(i32, (32, 32), 1)
    s_lower = (row_t < row).astype(bf16)

    def rank_and_count(m):
        mfp = m.astype(f32)
        cs = jnp.dot(mfp.astype(bf16), triu_incl, preferred_element_type=f32)
        rank_in_row = cs - mfp
        row_sums = jnp.sum(mfp, axis=1, keepdims=True)
        rs_b = jnp.broadcast_to(row_sums, (32, 128)).astype(bf16)
        row_off = jnp.dot(s_lower, rs_b, preferred_element_type=f32)
        n = jnp.sum(mfp).astype(i32)
        return rank_in_row + row_off, n

    rank_c, n_c = rank_and_count(mc)
    rank_i, n_i = rank_and_count(mi)
    rank_f, n_f = rank_and_count(mf)

    nb_c = (n_c + BLK - 1) // BLK
    nb_i = (n_i + BLK - 1) // BLK
    nb_f = (n_f + BLK - 1) // BLK
    base_i = nb_c
    base_f = nb_c + nb_i

    pos = jnp.where(
        mc, rank_c,
        jnp.where(mi, base_i.astype(f32) * BLK + rank_i,
                  base_f.astype(f32) * BLK + rank_f))
    pos_ref[...] = pos.astype(i32)

    ml = lax.broadcasted_iota(i32, (1, 128), 1)
    meta = (jnp.where(ml == 0, nb_c, 0) + jnp.where(ml == 1, nb_i, 0)
            + jnp.where(ml == 2, nb_f, 0) + jnp.where(ml == 4, base_i, 0)
            + jnp.where(ml == 5, base_f, 0))
    meta_ref[...] = meta


def _routing_call(s):
    return pl.pallas_call(
        _routing_body,
        grid=(1,),
        in_specs=[pl.BlockSpec((32, 128), lambda i: (0, 0))],
        out_specs=[pl.BlockSpec((32, 128), lambda i: (0, 0)),
                   pl.BlockSpec((1, 128), lambda i: (0, 0))],
        out_shape=[jax.ShapeDtypeStruct((32, 128), i32),
                   jax.ShapeDtypeStruct((1, 128), i32)],
    )(s.reshape(32, 128))


@functools.cache
def _sc_mesh():
    return plsc.VectorSubcoreMesh(core_axis_name="c", subcore_axis_name="s")


def _wid():
    return lax.axis_index("s") * 2 + lax.axis_index("c")


def _sc_scatter_body(x_hbm, pos_hbm, xs_hbm, idx_v, rows_v, sem):
    base = _wid() * ROWS_32
    for c in range(ROWS_32 // CHUNK):
        r0 = base + c * CHUNK
        pltpu.sync_copy(pos_hbm.at[pl.ds(r0, CHUNK)], idx_v)
        pltpu.sync_copy(x_hbm.at[pl.ds(r0, CHUNK)], rows_v)
        pltpu.async_copy(rows_v, xs_hbm.at[idx_v], sem).wait()


def _sc_scatter(x, pos):
    fn = pl.kernel(
        _sc_scatter_body,
        out_type=jax.ShapeDtypeStruct((XS_ROWS, H), f32),
        mesh=_sc_mesh(),
        scratch_types=[pltpu.VMEM((CHUNK,), i32),
                       pltpu.VMEM((CHUNK, H), f32),
                       pltpu.SemaphoreType.DMA],
    )
    return fn(x, pos)


def _sc_gather_body(ys_hbm, pos_hbm, out_hbm, idx_v, rows_v, sem):
    base = _wid() * ROWS_32
    for c in range(ROWS_32 // CHUNK):
        r0 = base + c * CHUNK
        pltpu.sync_copy(pos_hbm.at[pl.ds(r0, CHUNK)], idx_v)
        pltpu.async_copy(ys_hbm.at[idx_v], rows_v, sem).wait()
        pltpu.sync_copy(rows_v, out_hbm.at[pl.ds(r0, CHUNK)])


def _sc_gather(ys, pos):
    fn = pl.kernel(
        _sc_gather_body,
        out_type=jax.ShapeDtypeStruct((SEQ, H), f32),
        mesh=_sc_mesh(),
        scratch_types=[pltpu.VMEM((CHUNK,), i32),
                       pltpu.VMEM((CHUNK, H), f32),
                       pltpu.SemaphoreType.DMA],
    )
    return fn(ys, pos)


def _branch_body(k, meta_ref, x_ref, w_ref, b1_ref, wd_ref, b2_ref,
                 g_ref, bt_ref, *rest):
    o_ref = rest[-1]
    i = pl.program_id(0)
    nb = meta_ref[k]

    @pl.when(i < nb)
    def _():
        x = x_ref[...].astype(bf16)
        z = jnp.dot(x, w_ref[...], preferred_element_type=f32) + b1_ref[...]
        y = jnp.dot(z.astype(bf16), wd_ref[...],
                    preferred_element_type=f32) + b2_ref[...]
        mean = jnp.mean(y, axis=-1, keepdims=True)
        yc = y - mean
        var = jnp.mean(yc * yc, axis=-1, keepdims=True)
        o_ref[...] = yc * lax.rsqrt(var + 1e-5) * g_ref[...] + bt_ref[...]


def _branch_call(k, meta, xs, w, b1, wd, b2, gamma, beta, ys_in):
    d = w.shape[1]

    def blk_map(i, m):
        return (m[3 + k] + jnp.maximum(jnp.minimum(i, m[k] - 1), 0), 0)

    const2 = lambda i, m: (0, 0)
    in_specs = [
        pl.BlockSpec((BLK, H), blk_map),
        pl.BlockSpec((H, d), const2),
        pl.BlockSpec((1, d), const2),
        pl.BlockSpec((d, H), const2),
        pl.BlockSpec((1, H), const2),
        pl.BlockSpec((1, H), const2),
        pl.BlockSpec((1, H), const2),
    ]
    args = [meta, xs, w, b1, wd, b2, gamma, beta]
    aliases = {}
    if ys_in is not None:
        in_specs.append(pl.BlockSpec(memory_space=pl.ANY))
        args.append(ys_in)
        aliases = {8: 0}
    grid_spec = pltpu.PrefetchScalarGridSpec(
        num_scalar_prefetch=1,
        grid=(NBLK,),
        in_specs=in_specs,
        out_specs=pl.BlockSpec((BLK, H), blk_map),
    )
    return pl.pallas_call(
        functools.partial(_branch_body, k),
        grid_spec=grid_spec,
        out_shape=jax.ShapeDtypeStruct((XS_ROWS, H), f32),
        input_output_aliases=aliases,
    )(*args)


def kernel(hidden_states, importance_scores, Wc, bc, Wi, bi, Wf, bf,
           Wdc, bdc, Wdi, bdi, Wdf, bdf, gamma, beta):
    pos2d, meta2d = _routing_call(importance_scores)
    pos = pos2d.reshape(SEQ)
    meta = meta2d.reshape(128)

    xs = _sc_scatter(hidden_states, pos)

    g2 = gamma.reshape(1, H)
    bt2 = beta.reshape(1, H)
    ys = _branch_call(0, meta, xs, Wc.astype(bf16), bc.reshape(1, -1),
                      Wdc.astype(bf16), bdc.reshape(1, H), g2, bt2, None)
    ys = _branch_call(1, meta, xs, Wi.astype(bf16), bi.reshape(1, -1),
                      Wdi.astype(bf16), bdi.reshape(1, H), g2, bt2, ys)
    ys = _branch_call(2, meta, xs, Wf.astype(bf16), bf.reshape(1, -1),
                      Wdf.astype(bf16), bdf.reshape(1, H), g2, bt2, ys)

    return _sc_gather(ys, pos)
